# Initial kernel scaffold; baseline (speedup 1.0000x reference)
import jax
import jax.numpy as jnp
from jax.experimental import pallas as pl


def kernel(x, ei_temp_previous, ei_intersects, ei_represented_by, ei_coplanar, W_head, b_head, W1, b1, W2, b2, W3, b3, W4, b4, W_lin, b_lin):
    raise NotImplementedError("write your pallas kernel here")


# speedup vs baseline: 1.0158x; 1.0158x over previous
import functools

import jax
import jax.numpy as jnp
from jax.experimental import pallas as pl
from jax.experimental.pallas import tpu as pltpu

_N = 10000
_NPAD = 10240
_BN = 1024


def _mm_body(h_ref, w_ref, b_ref, o_ref):
    acc = jnp.dot(h_ref[...], w_ref[0], preferred_element_type=jnp.float32)
    o_ref[...] = (acc + b_ref[0])[None]


def _batched_mm(h, W, b):
    npad, k = h.shape
    nrel, _, hid = W.shape
    grid = (nrel, npad // _BN)
    b3 = b.reshape(nrel, 1, hid)
    return pl.pallas_call(
        _mm_body,
        grid=grid,
        in_specs=[
            pl.BlockSpec((_BN, k), lambda r, i: (i, 0)),
            pl.BlockSpec((1, k, hid), lambda r, i: (r, 0, 0)),
            pl.BlockSpec((1, 1, hid), lambda r, i: (r, 0, 0)),
        ],
        out_specs=pl.BlockSpec((1, _BN, hid), lambda r, i: (r, i, 0)),
        out_shape=jax.ShapeDtypeStruct((nrel, npad, hid), jnp.float32),
    )(h, W, b3)


def _lin_body(h_ref, w_ref, b_ref, o_ref):
    o_ref[...] = (
        jnp.dot(h_ref[...], w_ref[...], preferred_element_type=jnp.float32)
        + b_ref[...]
    )


def _final_linear(h, W, b):
    npad, k = h.shape
    hout = W.shape[1]
    return pl.pallas_call(
        _lin_body,
        grid=(npad // _BN,),
        in_specs=[
            pl.BlockSpec((_BN, k), lambda i: (i, 0)),
            pl.BlockSpec((k, hout), lambda i: (0, 0)),
            pl.BlockSpec((1, hout), lambda i: (0, 0)),
        ],
        out_specs=pl.BlockSpec((_BN, hout), lambda i: (i, 0)),
        out_shape=jax.ShapeDtypeStruct((npad, hout), jnp.float32),
    )(h, W, b.reshape(1, hout))


def _aggregate(y, eis, inv_cnt, n):
    out = None
    for i, agg in enumerate(('add', 'mean', 'mean', 'max')):
        src = eis[i][0]
        dst = eis[i][1]
        msg = y[i][src]
        if agg == 'add':
            a = jax.ops.segment_sum(msg, dst, num_segments=n)
        elif agg == 'mean':
            s = jax.ops.segment_sum(msg, dst, num_segments=n)
            a = s * inv_cnt[i][:, None]
        else:
            m = jax.ops.segment_max(msg, dst, num_segments=n)
            a = jnp.where(jnp.isfinite(m), m, 0.0)
        out = a if out is None else out + a
    return out


def kernel(x, ei_temp_previous, ei_intersects, ei_represented_by, ei_coplanar,
           W_head, b_head, W1, b1, W2, b2, W3, b3, W4, b4, W_lin, b_lin):
    eis = [ei_temp_previous, ei_intersects, ei_represented_by, ei_coplanar]
    n = x.shape[0]

    ones = jnp.ones((eis[0].shape[1],), dtype=jnp.float32)
    inv_cnt = []
    for i in range(4):
        cnt = jax.ops.segment_sum(ones, eis[i][1], num_segments=n)
        inv_cnt.append(1.0 / jnp.clip(cnt, 1.0))

    xpad = jnp.pad(x, ((0, _NPAD - n), (0, 0)))

    y = _batched_mm(xpad, W_head, b_head)
    h = jax.nn.relu(_aggregate(y[:, :n], eis, inv_cnt, n))
    for Wl, bl in ((W1, b1), (W2, b2), (W3, b3), (W4, b4)):
        hpad = jnp.pad(h, ((0, _NPAD - n), (0, 0)))
        y = _batched_mm(hpad, Wl, bl)
        h = jax.nn.relu(_aggregate(y[:, :n], eis, inv_cnt, n)) + h

    wlin_pad = jnp.pad(W_lin, ((0, 0), (0, 128 - W_lin.shape[1])))
    blin_pad = jnp.pad(b_lin, (0, 128 - b_lin.shape[0]))
    hpad = jnp.pad(h, ((0, _NPAD - n), (0, 0)))
    out = _final_linear(hpad, wlin_pad, blin_pad)
    return out[:n, : W_lin.shape[1]]


# speedup vs baseline: 1.7017x; 1.6752x over previous
import functools

import jax
import jax.numpy as jnp
from jax import lax
from jax.experimental import pallas as pl
from jax.experimental.pallas import tpu as pltpu
from jax.experimental.pallas import tpu_sc as plsc

_N = 10000
_NPAD = 10240
_BN = 1024
_E = 320000
_NTILES = 16
_EPT = _E // _NTILES
_CH = 80
_NPAIR = _EPT // (2 * _CH)
_RPT = _NPAD // _NTILES


def _mm_body(h_ref, w_ref, b_ref, o_ref):
    acc = jnp.dot(h_ref[...], w_ref[0], preferred_element_type=jnp.float32)
    o_ref[...] = (acc + b_ref[0])[None]


def _batched_mm(h, W, b):
    npad, k = h.shape
    nrel, _, hid = W.shape
    b3 = b.reshape(nrel, 1, hid)
    return pl.pallas_call(
        _mm_body,
        grid=(nrel, npad // _BN),
        in_specs=[
            pl.BlockSpec((_BN, k), lambda r, i: (i, 0)),
            pl.BlockSpec((1, k, hid), lambda r, i: (r, 0, 0)),
            pl.BlockSpec((1, 1, hid), lambda r, i: (r, 0, 0)),
        ],
        out_specs=pl.BlockSpec((1, _BN, hid), lambda r, i: (r, i, 0)),
        out_shape=jax.ShapeDtypeStruct((nrel, npad, hid), jnp.float32),
    )(h, W, b3)


def _lin_body(h_ref, w_ref, b_ref, o_ref):
    o_ref[...] = (
        jnp.dot(h_ref[...], w_ref[...], preferred_element_type=jnp.float32)
        + b_ref[...]
    )


def _final_linear(h, W, b):
    npad, k = h.shape
    hout = W.shape[1]
    return pl.pallas_call(
        _lin_body,
        grid=(npad // _BN,),
        in_specs=[
            pl.BlockSpec((_BN, k), lambda i: (i, 0)),
            pl.BlockSpec((k, hout), lambda i: (0, 0)),
            pl.BlockSpec((1, hout), lambda i: (0, 0)),
        ],
        out_specs=pl.BlockSpec((_BN, hout), lambda i: (i, 0)),
        out_shape=jax.ShapeDtypeStruct((npad, hout), jnp.float32),
    )(h, W, b.reshape(1, hout))


def _fill_vmem(ref, rows, cols, value):
    val = jnp.full((16,), value, dtype=jnp.float32)

    def body(i, carry):
        for j in range(cols // 16):
            ref[i, pl.ds(j * 16, 16)] = val
        return carry

    lax.fori_loop(0, rows, body, 0)


def _make_sc_agg(with_counts):
    mesh = plsc.VectorSubcoreMesh(core_axis_name="c", subcore_axis_name="s")

    out_type = [jax.ShapeDtypeStruct((2, _NPAD, 128), jnp.float32)
                for _ in range(3)]
    if with_counts:
        out_type += [jax.ShapeDtypeStruct((2, _NPAD, 16), jnp.float32)
                     for _ in range(2)]

    scratch = [
        pltpu.VMEM_SHARED((_NPAD, 128), jnp.float32),
        pltpu.VMEM((_CH, 128), jnp.float32),
        pltpu.VMEM((_CH, 128), jnp.float32),
        pltpu.VMEM((_CH,), jnp.int32),
        pltpu.VMEM((_CH,), jnp.int32),
        pltpu.VMEM((_CH,), jnp.int32),
        pltpu.VMEM((_CH,), jnp.int32),
        pltpu.VMEM((32, 128), jnp.float32),
        pltpu.SemaphoreType.DMA,
        pltpu.SemaphoreType.DMA,
    ]
    if with_counts:
        scratch += [
            pltpu.VMEM_SHARED((_NPAD, 16), jnp.float32),
            pltpu.VMEM((_RPT, 16), jnp.float32),
            pltpu.VMEM((_CH, 16), jnp.float32),
        ]

    def body(y0, y1, y2, s0, d0, s1, d1, s2, d2, *rest):
        if with_counts:
            (agg0, agg1, agg2, cnt1, cnt2, acc, g_a, g_b, src_a, src_b,
             dst_a, dst_b, zbuf, sem_a, sem_b,
             acc16, zbuf16, ones16) = rest
            cnts = (None, cnt1, cnt2)
        else:
            (agg0, agg1, agg2, acc, g_a, g_b, src_a, src_b,
             dst_a, dst_b, zbuf, sem_a, sem_b) = rest
            cnts = (None, None, None)

        c = lax.axis_index("c")
        s = lax.axis_index("s")
        row0 = s * _RPT
        ebase = s * _EPT

        _fill_vmem(zbuf, 32, 128, 0.0)
        if with_counts:
            _fill_vmem(zbuf16, _RPT, 16, 0.0)
            _fill_vmem(ones16, _CH, 16, 1.0)

        def relation_pass(y, esrc, edst, agg, cnt, count_this):
            for k in range(_RPT // 32):
                pltpu.sync_copy(zbuf, acc.at[pl.ds(row0 + k * 32, 32), :])
            if count_this:
                pltpu.sync_copy(zbuf16, acc16.at[pl.ds(row0, _RPT), :])
            plsc.subcore_barrier()

            def issue(off, srcb, dstb, gb, sem):
                pltpu.sync_copy(esrc.at[pl.ds(off, _CH)], srcb)
                pltpu.sync_copy(edst.at[pl.ds(off, _CH)], dstb)
                for j in range(_CH // 16):
                    sl = pl.ds(j * 16, 16)
                    srcb[sl] = srcb[sl] * 2 + c
                return pltpu.async_copy(y.at[srcb], gb, sem)

            def chunk_pair(i, carry):
                off = ebase + i * (2 * _CH)
                da = issue(off, src_a, dst_a, g_a, sem_a)
                db = issue(off + _CH, src_b, dst_b, g_b, sem_b)
                da.wait()
                pltpu.sync_copy(g_a, acc.at[dst_a], add=True)
                if count_this:
                    pltpu.sync_copy(ones16, acc16.at[dst_a], add=True)
                db.wait()
                pltpu.sync_copy(g_b, acc.at[dst_b], add=True)
                if count_this:
                    pltpu.sync_copy(ones16, acc16.at[dst_b], add=True)
                return carry

            lax.fori_loop(0, _NPAIR, chunk_pair, 0)
            plsc.subcore_barrier()

            rows = pl.ds(row0, _RPT)
            pltpu.sync_copy(acc.at[rows, :], agg.at[c, rows, :])
            if count_this:
                pltpu.sync_copy(acc16.at[rows, :], cnt.at[c, rows, :])
            plsc.subcore_barrier()

        relation_pass(y0, s0, d0, agg0, None, False)
        relation_pass(y1, s1, d1, agg1, cnts[1], with_counts)
        relation_pass(y2, s2, d2, agg2, cnts[2], with_counts)

    kern = pl.kernel(
        body,
        out_type=out_type,
        mesh=mesh,
        scratch_types=scratch,
        compiler_params=pltpu.CompilerParams(use_tc_tiling_on_sc=False),
    )
    return kern


_sc_agg_cache = {}


def _sc_agg(with_counts):
    if with_counts not in _sc_agg_cache:
        _sc_agg_cache[with_counts] = _make_sc_agg(with_counts)
    return _sc_agg_cache[with_counts]


def _combine_body(a0_ref, a1_ref, a2_ref, c1_ref, c2_ref, m_ref, hp_ref,
                  o_ref):
    a0 = jnp.concatenate([a0_ref[0], a0_ref[1]], axis=-1)
    a1 = jnp.concatenate([a1_ref[0], a1_ref[1]], axis=-1)
    a2 = jnp.concatenate([a2_ref[0], a2_ref[1]], axis=-1)
    c1 = jnp.maximum(c1_ref[0][:, 0:1], 1.0)
    c2 = jnp.maximum(c2_ref[0][:, 0:1], 1.0)
    m = m_ref[...]
    m = jnp.where(jnp.isfinite(m), m, 0.0)
    tot = a0 + a1 / c1 + a2 / c2 + m
    o_ref[...] = jnp.maximum(tot, 0.0) + hp_ref[...]


def _combine(a0, a1, a2, cnt1, cnt2, mx, hprev):
    spec_a = pl.BlockSpec((2, _BN, 128), lambda i: (0, i, 0))
    spec_c = pl.BlockSpec((2, _BN, 16), lambda i: (0, i, 0))
    spec_h = pl.BlockSpec((_BN, 256), lambda i: (i, 0))
    return pl.pallas_call(
        _combine_body,
        grid=(_NPAD // _BN,),
        in_specs=[spec_a, spec_a, spec_a, spec_c, spec_c, spec_h, spec_h],
        out_specs=spec_h,
        out_shape=jax.ShapeDtypeStruct((_NPAD, 256), jnp.float32),
    )(a0, a1, a2, cnt1, cnt2, mx, hprev)


def kernel(x, ei_temp_previous, ei_intersects, ei_represented_by, ei_coplanar,
           W_head, b_head, W1, b1, W2, b2, W3, b3, W4, b4, W_lin, b_lin):
    eis = [ei_temp_previous, ei_intersects, ei_represented_by, ei_coplanar]
    n = x.shape[0]
    e0, e1, e2, e3 = eis

    hpad = jnp.pad(x, ((0, _NPAD - n), (0, 0)))
    hprev = jnp.zeros((_NPAD, 256), jnp.float32)
    cnt1 = cnt2 = None
    Ws = ((W_head, b_head), (W1, b1), (W2, b2), (W3, b3), (W4, b4))

    for li, (Wl, bl) in enumerate(Ws):
        y = _batched_mm(hpad, Wl, bl)
        yr = [y[i].reshape(2 * _NPAD, 128) for i in range(3)]
        if li == 0:
            a0, a1, a2, cnt1, cnt2 = _sc_agg(True)(
                yr[0], yr[1], yr[2],
                e0[0], e0[1], e1[0], e1[1], e2[0], e2[1])
        else:
            a0, a1, a2 = _sc_agg(False)(
                yr[0], yr[1], yr[2],
                e0[0], e0[1], e1[0], e1[1], e2[0], e2[1])
        mx = jax.ops.segment_max(y[3][:n][e3[0]], e3[1], num_segments=n)
        mxpad = jnp.pad(mx, ((0, _NPAD - n), (0, 0)))
        h = _combine(a0, a1, a2, cnt1, cnt2, mxpad, hprev)
        hprev = h
        hpad = h

    wlin_pad = jnp.pad(W_lin, ((0, 0), (0, 128 - W_lin.shape[1])))
    blin_pad = jnp.pad(b_lin, (0, 128 - b_lin.shape[0]))
    out = _final_linear(hpad, wlin_pad, blin_pad)
    return out[:n, : W_lin.shape[1]]
